# Initial kernel scaffold; baseline (speedup 1.0000x reference)
import jax
import jax.numpy as jnp
from jax.experimental import pallas as pl


def kernel(item_seq, edge_index, edge_weight, embedding):
    raise NotImplementedError("write your pallas kernel here")


# speedup vs baseline: 3.8071x; 3.8071x over previous
import functools

import jax
import jax.numpy as jnp
from jax import lax
from jax.experimental import pallas as pl
from jax.experimental.pallas import tpu as pltpu
from jax.experimental.pallas import tpu_sc as plsc

N_NODES = 10000
N_PAD = 10240
D = 128
N_EDGES = 320000
SEQ_TOTAL = 4096 * 50

NC, NS = 2, 16
NW = NC * NS
EDGES_PER_W = N_EDGES // NW
ECHUNK = 80
N_ECHUNKS = EDGES_PER_W // ECHUNK
ROWS_PER_TILE = N_PAD // NS
SEQ_PER_W = SEQ_TOTAL // NW
GCHUNK = 80
N_GCHUNKS = SEQ_PER_W // GCHUNK

_mesh = plsc.VectorSubcoreMesh(core_axis_name="c", subcore_axis_name="s")


def _layer_body(src_hbm, dst_hbm, w_hbm, zeros_hbm, x_hbm, out_hbm,
                acc, idx_s, idx_d, w_v, rows_v):
    c = lax.axis_index("c")
    s = lax.axis_index("s")
    wid = c * NS + s
    r0 = s * ROWS_PER_TILE
    pltpu.sync_copy(zeros_hbm.at[pl.ds(r0, ROWS_PER_TILE)],
                    acc.at[pl.ds(r0, ROWS_PER_TILE)])
    plsc.subcore_barrier()

    def chunk_body(i, carry):
        base = wid * EDGES_PER_W + i * ECHUNK
        pltpu.sync_copy(src_hbm.at[pl.ds(base, ECHUNK)], idx_s)
        pltpu.sync_copy(dst_hbm.at[pl.ds(base, ECHUNK)], idx_d)
        pltpu.sync_copy(w_hbm.at[pl.ds(base, ECHUNK)], w_v)
        pltpu.sync_copy(x_hbm.at[idx_s], rows_v)

        def mul_body(j, cc):
            w16 = w_v[pl.ds(j * 16, 16)]
            for e in range(16):
                we = w16[e]
                row = j * 16 + e
                for g in range(8):
                    sl = pl.ds(g * 16, 16)
                    rows_v[row, sl] = rows_v[row, sl] * we
            return cc

        lax.fori_loop(0, ECHUNK // 16, mul_body, 0, unroll=False)
        pltpu.sync_copy(rows_v, acc.at[idx_d], add=True)
        return carry

    lax.fori_loop(0, N_ECHUNKS, chunk_body, 0, unroll=False)
    plsc.subcore_barrier()
    pltpu.sync_copy(acc.at[pl.ds(r0, ROWS_PER_TILE)],
                    out_hbm.at[c, pl.ds(r0, ROWS_PER_TILE)])


_layer_kernel = functools.partial(
    pl.kernel,
    out_type=jax.ShapeDtypeStruct((NC, N_PAD, D), jnp.float32),
    mesh=_mesh,
    scratch_types=[
        pltpu.VMEM_SHARED((N_PAD, D), jnp.float32),
        pltpu.VMEM((ECHUNK,), jnp.int32),
        pltpu.VMEM((ECHUNK,), jnp.int32),
        pltpu.VMEM((ECHUNK,), jnp.float32),
        pltpu.VMEM((ECHUNK, D), jnp.float32),
    ],
)(_layer_body)


def _gather_body(xf_hbm, seq_hbm, out_hbm, xf, idx_v, rows_v):
    c = lax.axis_index("c")
    s = lax.axis_index("s")
    wid = c * NS + s
    r0 = s * ROWS_PER_TILE
    pltpu.sync_copy(xf_hbm.at[pl.ds(r0, ROWS_PER_TILE)],
                    xf.at[pl.ds(r0, ROWS_PER_TILE)])
    plsc.subcore_barrier()

    def g_body(i, carry):
        base = wid * SEQ_PER_W + i * GCHUNK
        pltpu.sync_copy(seq_hbm.at[pl.ds(base, GCHUNK)], idx_v)
        pltpu.sync_copy(xf.at[idx_v], rows_v)
        pltpu.sync_copy(rows_v, out_hbm.at[pl.ds(base, GCHUNK)])
        return carry

    lax.fori_loop(0, N_GCHUNKS, g_body, 0, unroll=False)


_gather_kernel = functools.partial(
    pl.kernel,
    out_type=jax.ShapeDtypeStruct((SEQ_TOTAL, D), jnp.float32),
    mesh=_mesh,
    scratch_types=[
        pltpu.VMEM_SHARED((N_PAD, D), jnp.float32),
        pltpu.VMEM((GCHUNK,), jnp.int32),
        pltpu.VMEM((GCHUNK, D), jnp.float32),
    ],
)(_gather_body)


def _sum2_body(p_ref, o_ref):
    o_ref[...] = p_ref[0] + p_ref[1]


def _final_mean_body(emb_ref, x1_ref, p2_ref, o_ref):
    o_ref[...] = (emb_ref[...] + x1_ref[...] + p2_ref[0] + p2_ref[1]) * (
        jnp.float32(1.0 / 3.0))


_sum2 = pl.pallas_call(
    _sum2_body, out_shape=jax.ShapeDtypeStruct((N_PAD, D), jnp.float32))

_final_mean = pl.pallas_call(
    _final_mean_body, out_shape=jax.ShapeDtypeStruct((N_PAD, D), jnp.float32))


def kernel(item_seq, edge_index, edge_weight, embedding):
    src = edge_index[0]
    dst = edge_index[1]
    zeros = jnp.zeros((N_PAD, D), jnp.float32)
    emb_p = jnp.concatenate(
        [embedding, jnp.zeros((N_PAD - N_NODES, D), jnp.float32)], axis=0)
    p1 = _layer_kernel(src, dst, edge_weight, zeros, emb_p)
    x1 = _sum2(p1)
    p2 = _layer_kernel(src, dst, edge_weight, zeros, x1)
    xf = _final_mean(emb_p, x1, p2)
    out = _gather_kernel(xf, item_seq.reshape(-1))
    return out.reshape(item_seq.shape[0], item_seq.shape[1], D)


# speedup vs baseline: 7.2867x; 1.9140x over previous
import functools

import jax
import jax.numpy as jnp
from jax import lax
from jax.experimental import pallas as pl
from jax.experimental.pallas import tpu as pltpu
from jax.experimental.pallas import tpu_sc as plsc

N_NODES = 10000
N_PAD = 10240
D = 128
N_EDGES = 320000
SEQ_TOTAL = 4096 * 50

NC, NS = 2, 16
NW = NC * NS
EDGES_PER_W = N_EDGES // NW
ECHUNK = 80
N_ECHUNKS = EDGES_PER_W // ECHUNK
GSZ = 25
GEDGES = GSZ * ECHUNK
NGROUPS = N_ECHUNKS // GSZ
ROWS_PER_TILE = N_PAD // NS
ZROWS = 80
SEQ_PER_W = SEQ_TOTAL // NW
GCHUNK = 80
N_GCHUNKS = SEQ_PER_W // GCHUNK

_mesh = plsc.VectorSubcoreMesh(core_axis_name="c", subcore_axis_name="s")


def _layer_body(src_hbm, dst_hbm, w_hbm, x_hbm, out_hbm,
                acc, src_l, w_l, dst_a, dst_b, rows_a, rows_b,
                sem_a, sem_b, sem_da, sem_db):
    c = lax.axis_index("c")
    s = lax.axis_index("s")
    wid = c * NS + s
    r0 = s * ROWS_PER_TILE
    e0 = wid * EDGES_PER_W

    def zrow(i, cc):
        for g in range(D // 16):
            rows_a[i, pl.ds(g * 16, 16)] = jnp.zeros((16,), jnp.float32)
        return cc

    lax.fori_loop(0, ZROWS, zrow, 0, unroll=False)

    def zcp(k, cc):
        pltpu.sync_copy(rows_a, acc.at[pl.ds(r0 + k * ZROWS, ZROWS)])
        return cc

    lax.fori_loop(0, ROWS_PER_TILE // ZROWS, zcp, 0, unroll=False)
    plsc.subcore_barrier()

    def g_start(g, i, rows, dstc, sem, semd):
        pltpu.async_copy(x_hbm.at[src_l.at[pl.ds(i * ECHUNK, ECHUNK)]],
                         rows, sem)
        pltpu.async_copy(dst_hbm.at[pl.ds(e0 + g * GEDGES + i * ECHUNK,
                                          ECHUNK)], dstc, semd)

    def g_wait(g, i, rows, dstc, sem, semd):
        pltpu.make_async_copy(
            x_hbm.at[src_l.at[pl.ds(i * ECHUNK, ECHUNK)]], rows, sem).wait()
        pltpu.make_async_copy(
            dst_hbm.at[pl.ds(e0 + g * GEDGES + i * ECHUNK, ECHUNK)],
            dstc, semd).wait()

    def muls(i, rows):
        def mul_body(j, cc):
            w16 = w_l[pl.ds(i * ECHUNK + j * 16, 16)]
            for e in range(16):
                we = w16[e]
                row = j * 16 + e
                for g in range(D // 16):
                    sl = pl.ds(g * 16, 16)
                    rows[row, sl] = rows[row, sl] * we
            return cc

        lax.fori_loop(0, ECHUNK // 16, mul_body, 0, unroll=False)

    def scat(rows, dstc):
        pltpu.sync_copy(rows, acc.at[dstc], add=True)

    def group_body(g, cc):
        gb = e0 + g * GEDGES
        pltpu.sync_copy(src_hbm.at[pl.ds(gb, GEDGES)], src_l)
        pltpu.sync_copy(w_hbm.at[pl.ds(gb, GEDGES)], w_l)
        g_start(g, 0, rows_a, dst_a, sem_a, sem_da)

        def main_body(t, cc2):
            i0 = 2 * t
            g_wait(g, i0, rows_a, dst_a, sem_a, sem_da)
            g_start(g, i0 + 1, rows_b, dst_b, sem_b, sem_db)
            muls(i0, rows_a)
            scat(rows_a, dst_a)
            g_wait(g, i0 + 1, rows_b, dst_b, sem_b, sem_db)
            g_start(g, i0 + 2, rows_a, dst_a, sem_a, sem_da)
            muls(i0 + 1, rows_b)
            scat(rows_b, dst_b)
            return cc2

        lax.fori_loop(0, (GSZ - 1) // 2, main_body, 0, unroll=False)
        last = GSZ - 1
        g_wait(g, last, rows_a, dst_a, sem_a, sem_da)
        muls(last, rows_a)
        scat(rows_a, dst_a)
        return cc

    lax.fori_loop(0, NGROUPS, group_body, 0, unroll=False)

    plsc.subcore_barrier()
    pltpu.sync_copy(acc.at[pl.ds(r0, ROWS_PER_TILE)],
                    out_hbm.at[c, pl.ds(r0, ROWS_PER_TILE)])


def _make_layer():
    return functools.partial(
        pl.kernel,
        out_type=jax.ShapeDtypeStruct((NC, N_PAD, D), jnp.float32),
        mesh=_mesh,
        scratch_types=[
            pltpu.VMEM_SHARED((N_PAD, D), jnp.float32),
            pltpu.VMEM((GEDGES,), jnp.int32),
            pltpu.VMEM((GEDGES,), jnp.float32),
            pltpu.VMEM((ECHUNK,), jnp.int32),
            pltpu.VMEM((ECHUNK,), jnp.int32),
            pltpu.VMEM((ECHUNK, D), jnp.float32),
            pltpu.VMEM((ECHUNK, D), jnp.float32),
            pltpu.SemaphoreType.DMA,
            pltpu.SemaphoreType.DMA,
            pltpu.SemaphoreType.DMA,
            pltpu.SemaphoreType.DMA,
        ],
    )(_layer_body)


_layer1_kernel = _make_layer()
_layer2_kernel = _make_layer()


def _gather_body(xf_hbm, seq_hbm, out_hbm, xf, seq_l, rows_a, rows_b,
                 sem_a, sem_b):
    c = lax.axis_index("c")
    s = lax.axis_index("s")
    wid = c * NS + s
    r0 = s * ROWS_PER_TILE
    q0 = wid * SEQ_PER_W
    pltpu.sync_copy(xf_hbm.at[pl.ds(r0, ROWS_PER_TILE)],
                    xf.at[pl.ds(r0, ROWS_PER_TILE)])
    pltpu.sync_copy(seq_hbm.at[pl.ds(q0, SEQ_PER_W)], seq_l)
    plsc.subcore_barrier()

    def g_start(i, rows, sem):
        pltpu.async_copy(xf.at[seq_l.at[pl.ds(i * GCHUNK, GCHUNK)]], rows, sem)

    def g_wait(i, rows, sem):
        pltpu.make_async_copy(
            xf.at[seq_l.at[pl.ds(i * GCHUNK, GCHUNK)]], rows, sem).wait()

    def put(i, rows):
        pltpu.sync_copy(rows, out_hbm.at[pl.ds(q0 + i * GCHUNK, GCHUNK)])

    g_start(0, rows_a, sem_a)

    def main_body(t, cc):
        i0 = 2 * t
        g_wait(i0, rows_a, sem_a)
        g_start(i0 + 1, rows_b, sem_b)
        put(i0, rows_a)
        g_wait(i0 + 1, rows_b, sem_b)
        g_start(i0 + 2, rows_a, sem_a)
        put(i0 + 1, rows_b)
        return cc

    lax.fori_loop(0, (N_GCHUNKS - 2) // 2, main_body, 0, unroll=False)
    g_wait(N_GCHUNKS - 2, rows_a, sem_a)
    g_start(N_GCHUNKS - 1, rows_b, sem_b)
    put(N_GCHUNKS - 2, rows_a)
    g_wait(N_GCHUNKS - 1, rows_b, sem_b)
    put(N_GCHUNKS - 1, rows_b)


_gather_kernel = functools.partial(
    pl.kernel,
    out_type=jax.ShapeDtypeStruct((SEQ_TOTAL, D), jnp.float32),
    mesh=_mesh,
    scratch_types=[
        pltpu.VMEM_SHARED((N_PAD, D), jnp.float32),
        pltpu.VMEM((SEQ_PER_W,), jnp.int32),
        pltpu.VMEM((GCHUNK, D), jnp.float32),
        pltpu.VMEM((GCHUNK, D), jnp.float32),
        pltpu.SemaphoreType.DMA,
        pltpu.SemaphoreType.DMA,
    ],
)(_gather_body)


def _sum2_body(p_ref, o_ref):
    o_ref[...] = p_ref[0] + p_ref[1]


def _final_mean_body(emb_ref, x1_ref, p2_ref, o_ref):
    third = jnp.float32(1.0 / 3.0)
    top = (emb_ref[...] + x1_ref[pl.ds(0, N_NODES)] +
           p2_ref[0, pl.ds(0, N_NODES)] + p2_ref[1, pl.ds(0, N_NODES)]) * third
    o_ref[pl.ds(0, N_NODES)] = top
    o_ref[pl.ds(N_NODES, N_PAD - N_NODES)] = jnp.zeros(
        (N_PAD - N_NODES, D), jnp.float32)


_sum2 = pl.pallas_call(
    _sum2_body, out_shape=jax.ShapeDtypeStruct((N_PAD, D), jnp.float32))

_final_mean = pl.pallas_call(
    _final_mean_body, out_shape=jax.ShapeDtypeStruct((N_PAD, D), jnp.float32))


def kernel(item_seq, edge_index, edge_weight, embedding):
    src = edge_index[0]
    dst = edge_index[1]
    seq = item_seq.reshape(-1)
    p1 = _layer1_kernel(src, dst, edge_weight, embedding)
    x1 = _sum2(p1)
    p2 = _layer2_kernel(src, dst, edge_weight, x1)
    xf = _final_mean(embedding, x1, p2)
    out = _gather_kernel(xf, seq)
    return out.reshape(item_seq.shape[0], item_seq.shape[1], D)
